# Initial kernel scaffold; baseline (speedup 1.0000x reference)
import jax
import jax.numpy as jnp
from jax.experimental import pallas as pl


def kernel(x, edge_index, params):
    raise NotImplementedError("write your pallas kernel here")


# speedup vs baseline: 64.4410x; 64.4410x over previous
import functools

import jax
import jax.numpy as jnp
from jax import lax
from jax.experimental import pallas as pl
from jax.experimental.pallas import tpu as pltpu
from jax.experimental.pallas import tpu_sc as plsc

_N = 100000
_E = 1600000
_H = 32
_CHUNK = 2000
_NW = 32
_EPW = _E // _NW
_EPT = _E // 16
_BN = 2000
_CHUNK3 = 1000


def _sc_mesh():
    return plsc.VectorSubcoreMesh(core_axis_name="c", subcore_axis_name="s")


_SC_PARAMS = pltpu.CompilerParams(use_tc_tiling_on_sc=False)


def _deg_call(dst, ones_c, zeros_n):
    @functools.partial(
        pl.kernel,
        out_type=jax.ShapeDtypeStruct((2, _N), jnp.float32),
        mesh=_sc_mesh(),
        compiler_params=_SC_PARAMS,
        scratch_types=[
            pltpu.VMEM((_CHUNK,), jnp.int32),
            pltpu.VMEM((_CHUNK,), jnp.float32),
            pltpu.VMEM_SHARED((_N,), jnp.float32),
        ],
    )
    def deg_k(dst_hbm, ones_hbm, zeros_hbm, out_hbm, idx_v, ones_v, acc_sh):
        c = lax.axis_index("c")
        s = lax.axis_index("s")
        wid = s * 2 + c
        pltpu.sync_copy(ones_hbm, ones_v)

        @pl.when(s == 0)
        def _():
            pltpu.sync_copy(zeros_hbm, acc_sh)

        plsc.subcore_barrier()
        base = wid * _EPW

        def body(i, carry):
            off = base + i * _CHUNK
            pltpu.sync_copy(dst_hbm.at[pl.ds(off, _CHUNK)], idx_v)
            pltpu.sync_copy(ones_v, acc_sh.at[idx_v], add=True)
            return carry

        lax.fori_loop(0, _EPW // _CHUNK, body, 0)
        plsc.subcore_barrier()

        @pl.when(s == 0)
        def _():
            pltpu.sync_copy(acc_sh, out_hbm.at[c])

    return deg_k(dst, ones_c, zeros_n)


def _s1_call(src, dst, u, zeros_n):
    @functools.partial(
        pl.kernel,
        out_type=jax.ShapeDtypeStruct((2, _N), jnp.float32),
        mesh=_sc_mesh(),
        compiler_params=_SC_PARAMS,
        scratch_types=[
            pltpu.VMEM((_CHUNK,), jnp.int32),
            pltpu.VMEM((_CHUNK,), jnp.int32),
            pltpu.VMEM((_CHUNK,), jnp.float32),
            pltpu.VMEM_SHARED((_N,), jnp.float32),
            pltpu.SemaphoreType.DMA,
        ],
    )
    def s1_k(src_hbm, dst_hbm, u_hbm, zeros_hbm, out_hbm,
             idx_s, idx_d, vals_v, acc_sh, sem):
        c = lax.axis_index("c")
        s = lax.axis_index("s")
        wid = s * 2 + c

        @pl.when(s == 0)
        def _():
            pltpu.sync_copy(zeros_hbm, acc_sh)

        plsc.subcore_barrier()
        base = wid * _EPW

        def body(i, carry):
            off = base + i * _CHUNK
            pltpu.sync_copy(src_hbm.at[pl.ds(off, _CHUNK)], idx_s)
            pltpu.sync_copy(dst_hbm.at[pl.ds(off, _CHUNK)], idx_d)
            pltpu.async_copy(u_hbm.at[idx_s], vals_v, sem).wait()
            pltpu.sync_copy(vals_v, acc_sh.at[idx_d], add=True)
            return carry

        lax.fori_loop(0, _EPW // _CHUNK, body, 0)
        plsc.subcore_barrier()

        @pl.when(s == 0)
        def _():
            pltpu.sync_copy(acc_sh, out_hbm.at[c])

    return s1_k(src, dst, u, zeros_n)


def _s2_call(src, dst, u2a, u2b, zeros_n16):
    @functools.partial(
        pl.kernel,
        out_type=jax.ShapeDtypeStruct((2, _N, 16), jnp.float32),
        mesh=_sc_mesh(),
        compiler_params=_SC_PARAMS,
        scratch_types=[
            pltpu.VMEM((_CHUNK3,), jnp.int32),
            pltpu.VMEM((_CHUNK3,), jnp.int32),
            pltpu.VMEM((_CHUNK3, 16), jnp.float32),
            pltpu.VMEM_SHARED((_N, 16), jnp.float32),
            pltpu.SemaphoreType.DMA,
        ],
    )
    def s2_k(src_hbm, dst_hbm, u2a_hbm, u2b_hbm, zeros_hbm, out_hbm,
             idx_s, idx_d, rows_v, acc_sh, sem):
        c = lax.axis_index("c")
        s = lax.axis_index("s")

        @pl.when(s == 0)
        def _():
            pltpu.sync_copy(zeros_hbm, acc_sh)

        plsc.subcore_barrier()
        base = s * _EPT

        def body(i, carry):
            off = base + i * _CHUNK3
            pltpu.sync_copy(src_hbm.at[pl.ds(off, _CHUNK3)], idx_s)
            pltpu.sync_copy(dst_hbm.at[pl.ds(off, _CHUNK3)], idx_d)

            @pl.when(c == 0)
            def _():
                pltpu.async_copy(u2a_hbm.at[idx_s], rows_v, sem).wait()

            @pl.when(c == 1)
            def _():
                pltpu.async_copy(u2b_hbm.at[idx_s], rows_v, sem).wait()

            pltpu.sync_copy(rows_v, acc_sh.at[idx_d], add=True)
            return carry

        lax.fori_loop(0, _EPT // _CHUNK3, body, 0)
        plsc.subcore_barrier()

        @pl.when(s == 0)
        def _():
            pltpu.sync_copy(acc_sh, out_hbm.at[c])

    return s2_k(src, dst, u2a, u2b, zeros_n16)


def _tc_a(deg_parts, x0):
    def a_k(parts_ref, x_ref, dinv_ref, u_ref):
        deg = parts_ref[0] + parts_ref[1] + 1.0
        dinv = lax.rsqrt(deg)
        dinv_ref[:, :] = dinv
        u_ref[:, :] = dinv * x_ref[:, :]

    return pl.pallas_call(
        a_k,
        out_shape=[
            jax.ShapeDtypeStruct((800, 125), jnp.float32),
            jax.ShapeDtypeStruct((800, 125), jnp.float32),
        ],
    )(deg_parts.reshape(2, 800, 125), x0.reshape(800, 125))


def _tc_b(dinv, s1a, s1b, u, az1, cz1, ah1, ch1):
    def b_k(dinv_ref, s1a_ref, s1b_ref, u_ref, az_ref, czr, ah_ref, chr_,
            u2a_ref, u2b_ref, d16_ref):
        dinv = dinv_ref[:, :]
        y1 = dinv * (s1a_ref[:, :] + s1b_ref[:, :] + u_ref[:, :])
        pz = y1 * az_ref[:, :] + czr[:, :]
        ph = y1 * ah_ref[:, :] + chr_[:, :]
        h1 = jnp.maximum((1.0 - jax.nn.sigmoid(pz)) * jnp.tanh(ph), 0.0)
        u2 = dinv * h1
        u2a_ref[:, :] = u2[:, :16]
        u2b_ref[:, :] = u2[:, 16:]
        d16_ref[:, :] = jnp.broadcast_to(dinv, (_BN, 16))

    grid = _N // _BN
    col = pl.BlockSpec((_BN, 1), lambda i: (i, 0))
    wrow = pl.BlockSpec((1, _H), lambda i: (0, 0))
    half = pl.BlockSpec((_BN, 16), lambda i: (i, 0))
    return pl.pallas_call(
        b_k,
        grid=(grid,),
        in_specs=[col, col, col, col, wrow, wrow, wrow, wrow],
        out_specs=[half, half, half],
        out_shape=[
            jax.ShapeDtypeStruct((_N, 16), jnp.float32),
            jax.ShapeDtypeStruct((_N, 16), jnp.float32),
            jax.ShapeDtypeStruct((_N, 16), jnp.float32),
        ],
    )(dinv.reshape(_N, 1), s1a.reshape(_N, 1), s1b.reshape(_N, 1),
      u.reshape(_N, 1), az1, cz1, ah1, ch1)


def _tc_c(s2a, s2b, u2a, u2b, d16, Az2, cz2, Ah2, ch2, wo, bo):
    def c_k(s2a_ref, s2b_ref, u2a_ref, u2b_ref, d16_ref,
            az_ref, czr, ah_ref, chr_, wo_ref, bo_ref, out_ref):
        d16 = d16_ref[:, :]
        ya = d16 * (s2a_ref[:, :] + u2a_ref[:, :])
        yb = d16 * (s2b_ref[:, :] + u2b_ref[:, :])
        y2 = jnp.concatenate([ya, yb], axis=1)
        pz = jnp.dot(y2, az_ref[:, :], preferred_element_type=jnp.float32)
        ph = jnp.dot(y2, ah_ref[:, :], preferred_element_type=jnp.float32)
        gz = jax.nn.sigmoid(pz + czr[:, :])
        gh = jnp.tanh(ph + chr_[:, :])
        h2 = jnp.maximum((1.0 - gz) * gh, 0.0)
        out_ref[:, :] = (
            jnp.dot(h2, wo_ref[:, :], preferred_element_type=jnp.float32)
            + bo_ref[:, :])

    grid = _N // _BN
    half = pl.BlockSpec((_BN, 16), lambda i: (i, 0))
    wfull = pl.BlockSpec((_H, _H), lambda i: (0, 0))
    wrow = pl.BlockSpec((1, _H), lambda i: (0, 0))
    wcol = pl.BlockSpec((_H, 1), lambda i: (0, 0))
    wone = pl.BlockSpec((1, 1), lambda i: (0, 0))
    col = pl.BlockSpec((_BN, 1), lambda i: (i, 0))
    return pl.pallas_call(
        c_k,
        grid=(grid,),
        in_specs=[half, half, half, half, half,
                  wfull, wrow, wfull, wrow, wcol, wone],
        out_specs=col,
        out_shape=jax.ShapeDtypeStruct((_N, 1), jnp.float32),
    )(s2a, s2b, u2a, u2b, d16, Az2, cz2, Ah2, ch2, wo, bo)


def kernel(x, edge_index, params):
    p = params
    src = edge_index[0]
    dst = edge_index[1]
    x0 = x[0, :, 0, 0]

    az1 = p['Wc_z1'] @ p['Wl_z1'][:_H]
    cz1 = (p['bc_z1'] @ p['Wl_z1'][:_H] + p['bl_z1'])[None]
    ah1 = p['Wc_h1'] @ p['Wl_h1'][:_H]
    ch1 = (p['bc_h1'] @ p['Wl_h1'][:_H] + p['bl_h1'])[None]
    Az2 = p['Wc_z2'] @ p['Wl_z2'][:_H]
    cz2 = (p['bc_z2'] @ p['Wl_z2'][:_H] + p['bl_z2'])[None]
    Ah2 = p['Wc_h2'] @ p['Wl_h2'][:_H]
    ch2 = (p['bc_h2'] @ p['Wl_h2'][:_H] + p['bl_h2'])[None]
    wo = p['W_out']
    bo = p['b_out'][None]

    ones_c = jnp.ones((_CHUNK,), jnp.float32)
    zeros_n = jnp.zeros((_N,), jnp.float32)
    zeros_n16 = jnp.zeros((_N, 16), jnp.float32)

    deg_parts = _deg_call(dst, ones_c, zeros_n)
    dinv, u = _tc_a(deg_parts, x0)
    dinv = dinv.reshape(_N)
    u = u.reshape(_N)
    s1 = _s1_call(src, dst, u, zeros_n)
    u2a, u2b, d16 = _tc_b(dinv, s1[0], s1[1], u, az1, cz1, ah1, ch1)
    s2 = _s2_call(src, dst, u2a, u2b, zeros_n16)
    out = _tc_c(s2[0], s2[1], u2a, u2b, d16, Az2, cz2, Ah2, ch2, wo, bo)
    return out.reshape(1, _N, 1)
